# Initial kernel scaffold; baseline (speedup 1.0000x reference)
import functools

import jax
import jax.numpy as jnp
from jax import lax
from jax.experimental import pallas as pl
from jax.experimental.pallas import tpu as pltpu
from jax.experimental.pallas import tpu_sc as plsc

N = 10000
E = 320000
G = 64

NC, NS = 2, 16
NW = NC * NS
EPW = E // NW
CHUNK = 80
NCHUNK = EPW // CHUNK
RPS = N // NS
ZROWS = 125


def _sc_segment_sum(h, src_r, dst_r):
    mesh = plsc.VectorSubcoreMesh(core_axis_name="c", subcore_axis_name="s")

    @functools.partial(
        pl.kernel,
        out_type=jax.ShapeDtypeStruct((NC, N, 128), jnp.float32),
        mesh=mesh,
        scratch_types=[
            pltpu.VMEM((NCHUNK, CHUNK), jnp.int32),
            pltpu.VMEM((NCHUNK, CHUNK), jnp.int32),
            pltpu.VMEM((CHUNK, 128), jnp.float32),
            pltpu.VMEM((ZROWS, 128), jnp.float32),
            pltpu.VMEM_SHARED((N, 128), jnp.float32),
        ],
    )
    def k(h_hbm, src_hbm, dst_hbm, out_hbm, src_v, dst_v, rows_v, zero_v, agg_sh):
        cid = lax.axis_index("c")
        sid = lax.axis_index("s")
        wid = cid * NS + sid

        z16 = jnp.zeros((1, 16), jnp.float32)

        @pl.loop(0, ZROWS)
        def _(r):
            @pl.loop(0, 128, step=16)
            def _(c):
                zero_v.at[pl.ds(r, 1), pl.ds(c, 16)][...] = z16

        @pl.loop(0, RPS, step=ZROWS)
        def _(r0):
            pltpu.sync_copy(zero_v, agg_sh.at[pl.ds(sid * RPS + r0, ZROWS)])

        pltpu.sync_copy(src_hbm.at[wid], src_v)
        pltpu.sync_copy(dst_hbm.at[wid], dst_v)

        plsc.subcore_barrier()

        @pl.loop(0, NCHUNK)
        def _(j):
            pltpu.sync_copy(h_hbm.at[src_v.at[j]], rows_v)
            pltpu.sync_copy(rows_v, agg_sh.at[dst_v.at[j]], add=True)

        plsc.subcore_barrier()

        pltpu.sync_copy(agg_sh.at[pl.ds(sid * RPS, RPS)],
                        out_hbm.at[cid, pl.ds(sid * RPS, RPS)])

    return k(h, src_r, dst_r)


def _tc_affine(parts, b, relu):
    dout = b.shape[0]
    blk = 1000
    b2 = b.reshape(1, dout)
    nparts = len(parts)

    def body(*refs):
        o_ref = refs[-1]
        b_ref = refs[-2]
        acc = jnp.broadcast_to(b_ref[...], (blk, dout))
        for i in range(nparts):
            a = refs[2 * i][...]
            w = refs[2 * i + 1][...]
            acc = acc + lax.dot_general(a, w, (((1,), (0,)), ((), ())),
                                        precision=lax.Precision.HIGHEST,
                                        preferred_element_type=jnp.float32)
        o_ref[...] = jnp.maximum(acc, 0.0) if relu else acc

    in_specs = []
    args = []
    for a, w in parts:
        kk = a.shape[1]
        in_specs.append(pl.BlockSpec((blk, kk), lambda i: (i, 0)))
        in_specs.append(pl.BlockSpec((kk, dout), lambda i: (0, 0)))
        args.extend([a, w])
    in_specs.append(pl.BlockSpec((1, dout), lambda i: (0, 0)))
    args.append(b2)

    return pl.pallas_call(
        body,
        grid=(N // blk,),
        in_specs=in_specs,
        out_specs=pl.BlockSpec((blk, dout), lambda i: (i, 0)),
        out_shape=jax.ShapeDtypeStruct((N, dout), jnp.float32),
    )(*args)


def _tc_pool(h3, batch3):
    blk = 1000
    nb = N // blk
    dout = h3.shape[1]

    def body(h_ref, b_ref, o_ref, sums, cnts):
        i = pl.program_id(0)
        bb = b_ref[0, 0, :]
        oh = (bb[:, None] == lax.broadcasted_iota(jnp.int32, (blk, G), 1))
        oh = oh.astype(jnp.float32)
        psum = lax.dot_general(oh, h_ref[...], (((0,), (0,)), ((), ())),
                               precision=lax.Precision.HIGHEST,
                               preferred_element_type=jnp.float32)
        pcnt = jnp.sum(oh, axis=0).reshape(1, G)

        @pl.when(i == 0)
        def _():
            sums[...] = psum
            cnts[...] = pcnt

        @pl.when(i > 0)
        def _():
            sums[...] += psum
            cnts[...] += pcnt

        @pl.when(i == nb - 1)
        def _():
            cnt = jnp.maximum(cnts[...], 1.0).reshape(G, 1)
            pooled = sums[...] / cnt
            nrm = jnp.sqrt(jnp.sum(pooled * pooled, axis=1, keepdims=True))
            o_ref[...] = pooled / jnp.maximum(nrm, 1e-12)

    return pl.pallas_call(
        body,
        grid=(nb,),
        in_specs=[pl.BlockSpec((blk, dout), lambda i: (i, 0)),
                  pl.BlockSpec((1, 1, blk), lambda i: (i, 0, 0))],
        out_specs=pl.BlockSpec((G, dout), lambda i: (0, 0)),
        out_shape=jax.ShapeDtypeStruct((G, dout), jnp.float32),
        scratch_shapes=[pltpu.VMEM((G, dout), jnp.float32),
                        pltpu.VMEM((1, G), jnp.float32)],
    )(h3, batch3)


def kernel(x, edge_index, batch, W1_rel, b1, W1_root, W2_rel, b2, W2_root,
           W3_rel, b3, W3_root):
    src_r = edge_index[0].reshape(NW, NCHUNK, CHUNK)
    dst_r = edge_index[1].reshape(NW, NCHUNK, CHUNK)
    batch3 = batch.reshape(N // 1000, 1, 1000)

    a1 = _sc_segment_sum(x, src_r, dst_r)
    h1 = _tc_affine([(a1[0], W1_rel), (a1[1], W1_rel), (x, W1_root)], b1, True)

    a2 = _sc_segment_sum(h1, src_r, dst_r)
    h2 = _tc_affine([(a2[0], W2_rel), (a2[1], W2_rel), (h1, W2_root)], b2, True)

    a3a = _sc_segment_sum(h2[:, :128], src_r, dst_r)
    a3b = _sc_segment_sum(h2[:, 128:], src_r, dst_r)
    h3 = _tc_affine([(a3a[0], W3_rel[:128]), (a3a[1], W3_rel[:128]),
                     (a3b[0], W3_rel[128:]), (a3b[1], W3_rel[128:]),
                     (h2, W3_root)], b3, False)

    return _tc_pool(h3, batch3)


# speedup vs baseline: 5.0628x; 5.0628x over previous
import functools

import jax
import jax.numpy as jnp
from jax import lax
from jax.experimental import pallas as pl
from jax.experimental.pallas import tpu as pltpu
from jax.experimental.pallas import tpu_sc as plsc

N = 10000
E = 320000
G = 64

NC, NS = 2, 16
NW = NC * NS
EPW = E // NW
CHUNK = 80
NCHUNK = EPW // CHUNK
NPAD = 10240
RPS = NPAD // NS
ZROWS = 16


def _sc_segment_sum(h, src_r, dst_r):
    mesh = plsc.VectorSubcoreMesh(core_axis_name="c", subcore_axis_name="s")

    @functools.partial(
        pl.kernel,
        out_type=jax.ShapeDtypeStruct((NC, NPAD, 128), jnp.float32),
        mesh=mesh,
        scratch_types=[
            pltpu.VMEM((NCHUNK, CHUNK), jnp.int32),
            pltpu.VMEM((NCHUNK, CHUNK), jnp.int32),
            pltpu.VMEM((CHUNK, 128), jnp.float32),
            pltpu.VMEM((ZROWS, 128), jnp.float32),
            pltpu.VMEM_SHARED((NPAD, 128), jnp.float32),
        ],
    )
    def k(h_hbm, src_hbm, dst_hbm, out_hbm, src_v, dst_v, rows_v, zero_v, agg_sh):
        cid = lax.axis_index("c")
        sid = lax.axis_index("s")
        wid = cid * NS + sid

        z16 = jnp.zeros((1, 16), jnp.float32)

        @pl.loop(0, ZROWS)
        def _(r):
            @pl.loop(0, 128, step=16)
            def _(c):
                zero_v.at[pl.ds(r, 1), pl.ds(c, 16)][...] = z16

        @pl.loop(0, RPS, step=ZROWS)
        def _(r0):
            pltpu.sync_copy(zero_v, agg_sh.at[pl.ds(sid * RPS + r0, ZROWS)])

        pltpu.sync_copy(src_hbm.at[wid], src_v)
        pltpu.sync_copy(dst_hbm.at[wid], dst_v)

        plsc.subcore_barrier()

        @pl.loop(0, NCHUNK)
        def _(j):
            pltpu.sync_copy(h_hbm.at[src_v.at[j]], rows_v)
            pltpu.sync_copy(rows_v, agg_sh.at[dst_v.at[j]], add=True)

        plsc.subcore_barrier()

        pltpu.sync_copy(agg_sh.at[pl.ds(sid * RPS, RPS)],
                        out_hbm.at[cid, pl.ds(sid * RPS, RPS)])

    return k(h, src_r, dst_r)


def _tc_affine(parts, b, relu, out_splits=None):
    dout = b.shape[0]
    blk = 1000
    b2 = b.reshape(1, dout)
    nparts = len(parts)
    splits = out_splits or [dout]
    assert sum(splits) == dout

    def body(*refs):
        o_refs = refs[nparts * 2 + 1:]
        b_ref = refs[nparts * 2]
        acc = jnp.broadcast_to(b_ref[...], (blk, dout))
        for i in range(nparts):
            a = refs[2 * i][...]
            w = refs[2 * i + 1][...]
            acc = acc + lax.dot_general(a, w, (((1,), (0,)), ((), ())),
                                        precision=lax.Precision.HIGHEST,
                                        preferred_element_type=jnp.float32)
        if relu:
            acc = jnp.maximum(acc, 0.0)
        c0 = 0
        for o_ref, w in zip(o_refs, splits):
            o_ref[...] = acc[:, c0:c0 + w]
            c0 += w

    in_specs = []
    args = []
    for a, w in parts:
        kk = a.shape[1]
        in_specs.append(pl.BlockSpec((blk, kk), lambda i: (i, 0)))
        in_specs.append(pl.BlockSpec((kk, dout), lambda i: (0, 0)))
        args.extend([a, w])
    in_specs.append(pl.BlockSpec((1, dout), lambda i: (0, 0)))
    args.append(b2)

    out = pl.pallas_call(
        body,
        grid=(N // blk,),
        in_specs=in_specs,
        out_specs=[pl.BlockSpec((blk, w), lambda i: (i, 0)) for w in splits],
        out_shape=[jax.ShapeDtypeStruct((N, w), jnp.float32) for w in splits],
    )(*args)
    return out[0] if out_splits is None else out


def _tc_pool(h3, batch3):
    blk = 1000
    nb = N // blk
    dout = h3.shape[1]

    def body(h_ref, b_ref, o_ref, sums, cnts):
        i = pl.program_id(0)
        bb = b_ref[0, 0, :]
        oh = (bb[:, None] == lax.broadcasted_iota(jnp.int32, (blk, G), 1))
        oh = oh.astype(jnp.float32)
        psum = lax.dot_general(oh, h_ref[...], (((0,), (0,)), ((), ())),
                               precision=lax.Precision.HIGHEST,
                               preferred_element_type=jnp.float32)
        pcnt = jnp.sum(oh, axis=0).reshape(1, G)

        @pl.when(i == 0)
        def _():
            sums[...] = psum
            cnts[...] = pcnt

        @pl.when(i > 0)
        def _():
            sums[...] += psum
            cnts[...] += pcnt

        @pl.when(i == nb - 1)
        def _():
            cnt = jnp.maximum(cnts[...], 1.0).reshape(G, 1)
            pooled = sums[...] / cnt
            nrm = jnp.sqrt(jnp.sum(pooled * pooled, axis=1, keepdims=True))
            o_ref[...] = pooled / jnp.maximum(nrm, 1e-12)

    return pl.pallas_call(
        body,
        grid=(nb,),
        in_specs=[pl.BlockSpec((blk, dout), lambda i: (i, 0)),
                  pl.BlockSpec((1, 1, blk), lambda i: (i, 0, 0))],
        out_specs=pl.BlockSpec((G, dout), lambda i: (0, 0)),
        out_shape=jax.ShapeDtypeStruct((G, dout), jnp.float32),
        scratch_shapes=[pltpu.VMEM((G, dout), jnp.float32),
                        pltpu.VMEM((1, G), jnp.float32)],
    )(h3, batch3)


def kernel(x, edge_index, batch, W1_rel, b1, W1_root, W2_rel, b2, W2_root,
           W3_rel, b3, W3_root):
    src_r = edge_index[0].reshape(NW, NCHUNK, CHUNK)
    dst_r = edge_index[1].reshape(NW, NCHUNK, CHUNK)
    batch3 = batch.reshape(N // 1000, 1, 1000)

    a1 = _sc_segment_sum(x, src_r, dst_r)
    h1 = _tc_affine([(a1[0], W1_rel), (a1[1], W1_rel), (x, W1_root)], b1, True)

    a2 = _sc_segment_sum(h1, src_r, dst_r)
    h2a, h2b = _tc_affine([(a2[0], W2_rel), (a2[1], W2_rel), (h1, W2_root)],
                          b2, True, out_splits=[128, 128])

    a3a = _sc_segment_sum(h2a, src_r, dst_r)
    a3b = _sc_segment_sum(h2b, src_r, dst_r)
    h3 = _tc_affine([(a3a[0], W3_rel[:128]), (a3a[1], W3_rel[:128]),
                     (a3b[0], W3_rel[128:]), (a3b[1], W3_rel[128:]),
                     (h2a, W3_root[:128]), (h2b, W3_root[128:])], b3, False)

    return _tc_pool(h3, batch3)


# speedup vs baseline: 9.0913x; 1.7957x over previous
import dataclasses
import functools

import jax
import jax.numpy as jnp
from jax import lax
from jax.experimental import pallas as pl
from jax.experimental.pallas import tpu as pltpu
from jax.experimental.pallas import tpu_sc as plsc

N = 10000
E = 320000
G = 64

NC, NS = 2, 16
NW = NC * NS
EPW = E // NW
CHUNK = 80
NCHUNK = EPW // CHUNK
NPAD = 10240
RPS = NPAD // NS
ZROWS = 16


def _sc_segment_sum(h, src_r, dst_r):
    mesh = plsc.VectorSubcoreMesh(core_axis_name="c", subcore_axis_name="s")

    @functools.partial(
        pl.kernel,
        out_type=jax.ShapeDtypeStruct((NC, NPAD, 128), jnp.float32),
        mesh=mesh,
        scratch_types=[
            pltpu.VMEM((NCHUNK, CHUNK), jnp.int32),
            pltpu.VMEM((NCHUNK, CHUNK), jnp.int32),
            pltpu.VMEM((CHUNK, 128), jnp.float32),
            pltpu.VMEM((ZROWS, 128), jnp.float32),
            pltpu.VMEM_SHARED((NPAD, 128), jnp.float32),
        ],
    )
    def k(h_hbm, src_hbm, dst_hbm, out_hbm, src_v, dst_v, rows_v, zero_v, agg_sh):
        cid = lax.axis_index("c")
        sid = lax.axis_index("s")
        wid = cid * NS + sid

        z16 = jnp.zeros((1, 16), jnp.float32)

        @pl.loop(0, ZROWS)
        def _(r):
            @pl.loop(0, 128, step=16)
            def _(c):
                zero_v.at[pl.ds(r, 1), pl.ds(c, 16)][...] = z16

        @pl.loop(0, RPS, step=ZROWS)
        def _(r0):
            pltpu.sync_copy(zero_v, agg_sh.at[pl.ds(sid * RPS + r0, ZROWS)])

        pltpu.sync_copy(src_hbm.at[wid], src_v)
        pltpu.sync_copy(dst_hbm.at[wid], dst_v)

        plsc.subcore_barrier()

        @pl.loop(0, NCHUNK)
        def _(j):
            pltpu.sync_copy(h_hbm.at[src_v.at[j]], rows_v)
            pltpu.sync_copy(rows_v, agg_sh.at[dst_v.at[j]], add=True)

        plsc.subcore_barrier()

        pltpu.sync_copy(agg_sh.at[pl.ds(sid * RPS, RPS)],
                        out_hbm.at[cid, pl.ds(sid * RPS, RPS)])

    return k(h, src_r, dst_r)


def _sc_hist(src_r, dst_f, batch, zeros):
    mesh = plsc.VectorSubcoreMesh(core_axis_name="c", subcore_axis_name="s")
    cp = pltpu.CompilerParams()
    if "needs_layout_passes" in pltpu.CompilerParams.__dataclass_fields__:
        cp = dataclasses.replace(cp, needs_layout_passes=False)

    @functools.partial(
        pl.kernel,
        out_type=jax.ShapeDtypeStruct((NC, NPAD, 128), jnp.float32),
        mesh=mesh,
        compiler_params=cp,
        scratch_types=[
            pltpu.VMEM((NCHUNK, CHUNK), jnp.int32),
            pltpu.VMEM((EPW,), jnp.int32),
            pltpu.VMEM((N,), jnp.int32),
            pltpu.VMEM((CHUNK, 128), jnp.float32),
            pltpu.VMEM_SHARED((NPAD, 128), jnp.float32),
        ],
    )
    def k(src_hbm, dst_hbm, batch_hbm, zeros_hbm, out_hbm,
          src_v, dst_v, batch_v, oh_v, w_sh):
        cid = lax.axis_index("c")
        sid = lax.axis_index("s")
        wid = cid * NS + sid

        pltpu.sync_copy(zeros_hbm.at[pl.ds(0, CHUNK)], oh_v)

        @pl.loop(0, RPS, step=CHUNK)
        def _(r0):
            pltpu.sync_copy(oh_v, w_sh.at[pl.ds(sid * RPS + r0, CHUNK)])

        pltpu.sync_copy(src_hbm.at[wid], src_v)
        pltpu.sync_copy(dst_hbm.at[wid], dst_v)
        pltpu.sync_copy(batch_hbm, batch_v)

        plsc.subcore_barrier()

        ones16 = jnp.ones((16,), jnp.float32)
        zeros16 = jnp.zeros((16,), jnp.float32)
        iota16 = lax.iota(jnp.int32, 16)

        @pl.loop(0, NCHUNK)
        def _(j):
            cols = []
            for k16 in range(CHUNK // 16):
                off = j * CHUNK + 16 * k16
                d16 = dst_v.at[pl.ds(off, 16)][...]
                g16 = plsc.load_gather(batch_v, [d16])
                rows = iota16 + 16 * k16
                plsc.addupdate_scatter(oh_v, [rows, g16], ones16)
                cols.append((rows, g16))
            pltpu.sync_copy(oh_v, w_sh.at[src_v.at[j]], add=True)
            for rows, g16 in cols:
                plsc.store_scatter(oh_v, [rows, g16], zeros16)

        plsc.subcore_barrier()

        pltpu.sync_copy(w_sh.at[pl.ds(sid * RPS, RPS)],
                        out_hbm.at[cid, pl.ds(sid * RPS, RPS)])

    return k(src_r, dst_f, batch, zeros)


def _tc_affine(parts, b, relu, out_splits=None):
    dout = b.shape[0]
    blk = 1000
    b2 = b.reshape(1, dout)
    nparts = len(parts)
    splits = out_splits or [dout]
    assert sum(splits) == dout

    def body(*refs):
        o_refs = refs[nparts * 2 + 1:]
        b_ref = refs[nparts * 2]
        acc = jnp.broadcast_to(b_ref[...], (blk, dout))
        for i in range(nparts):
            a = refs[2 * i][...]
            w = refs[2 * i + 1][...]
            acc = acc + lax.dot_general(a, w, (((1,), (0,)), ((), ())),
                                        precision=lax.Precision.HIGHEST,
                                        preferred_element_type=jnp.float32)
        if relu:
            acc = jnp.maximum(acc, 0.0)
        c0 = 0
        for o_ref, w in zip(o_refs, splits):
            o_ref[...] = acc[:, c0:c0 + w]
            c0 += w

    in_specs = []
    args = []
    for a, w in parts:
        kk = a.shape[1]
        in_specs.append(pl.BlockSpec((blk, kk), lambda i: (i, 0)))
        in_specs.append(pl.BlockSpec((kk, dout), lambda i: (0, 0)))
        args.extend([a, w])
    in_specs.append(pl.BlockSpec((1, dout), lambda i: (0, 0)))
    args.append(b2)

    out = pl.pallas_call(
        body,
        grid=(N // blk,),
        in_specs=in_specs,
        out_specs=[pl.BlockSpec((blk, w), lambda i: (i, 0)) for w in splits],
        out_shape=[jax.ShapeDtypeStruct((N, w), jnp.float32) for w in splits],
    )(*args)
    return out[0] if out_splits is None else out


def _tc_pool_fused(h2a, h2b, w0, w1, batch3, W3_rel, W3_root, b3):
    blk = 1000
    nb = N // blk

    def body(ha_ref, hb_ref, w0_ref, w1_ref, b_ref, wr_ref, wt_ref, b3_ref,
             o_ref, S, B, cnts):
        i = pl.program_id(0)
        bb = b_ref[0, 0, :]
        oh = (bb[:, None] == lax.broadcasted_iota(jnp.int32, (blk, G), 1))
        oh = oh.astype(jnp.float32)
        ws = w0_ref[...][:, :G] + w1_ref[...][:, :G]

        def mmT(lhs, rhs):
            return lax.dot_general(lhs, rhs, (((0,), (0,)), ((), ())),
                                   precision=lax.Precision.HIGHEST,
                                   preferred_element_type=jnp.float32)

        sa = mmT(oh, ha_ref[...])
        sb = mmT(oh, hb_ref[...])
        ba = mmT(ws, ha_ref[...])
        bb2 = mmT(ws, hb_ref[...])
        pcnt = jnp.sum(oh, axis=0).reshape(1, G)

        @pl.when(i == 0)
        def _():
            S[:, :128] = sa
            S[:, 128:] = sb
            B[:, :128] = ba
            B[:, 128:] = bb2
            cnts[...] = pcnt

        @pl.when(i > 0)
        def _():
            S[:, :128] += sa
            S[:, 128:] += sb
            B[:, :128] += ba
            B[:, 128:] += bb2
            cnts[...] += pcnt

        @pl.when(i == nb - 1)
        def _():
            cnt = cnts[...].reshape(G, 1)
            psum = lax.dot_general(B[...], wr_ref[...], (((1,), (0,)), ((), ())),
                                   precision=lax.Precision.HIGHEST,
                                   preferred_element_type=jnp.float32)
            psum += lax.dot_general(S[...], wt_ref[...], (((1,), (0,)), ((), ())),
                                    precision=lax.Precision.HIGHEST,
                                    preferred_element_type=jnp.float32)
            psum += cnt * b3_ref[...]
            pooled = psum / jnp.maximum(cnt, 1.0)
            nrm = jnp.sqrt(jnp.sum(pooled * pooled, axis=1, keepdims=True))
            o_ref[...] = pooled / jnp.maximum(nrm, 1e-12)

    return pl.pallas_call(
        body,
        grid=(nb,),
        in_specs=[pl.BlockSpec((blk, 128), lambda i: (i, 0)),
                  pl.BlockSpec((blk, 128), lambda i: (i, 0)),
                  pl.BlockSpec((blk, 128), lambda i: (i, 0)),
                  pl.BlockSpec((blk, 128), lambda i: (i, 0)),
                  pl.BlockSpec((1, 1, blk), lambda i: (i, 0, 0)),
                  pl.BlockSpec((256, 512), lambda i: (0, 0)),
                  pl.BlockSpec((256, 512), lambda i: (0, 0)),
                  pl.BlockSpec((1, 512), lambda i: (0, 0))],
        out_specs=pl.BlockSpec((G, 512), lambda i: (0, 0)),
        out_shape=jax.ShapeDtypeStruct((G, 512), jnp.float32),
        scratch_shapes=[pltpu.VMEM((G, 256), jnp.float32),
                        pltpu.VMEM((G, 256), jnp.float32),
                        pltpu.VMEM((1, G), jnp.float32)],
    )(h2a, h2b, w0, w1, batch3, W3_rel, W3_root, b3.reshape(1, 512))


def kernel(x, edge_index, batch, W1_rel, b1, W1_root, W2_rel, b2, W2_root,
           W3_rel, b3, W3_root):
    src_r = edge_index[0].reshape(NW, NCHUNK, CHUNK)
    dst_r = edge_index[1].reshape(NW, NCHUNK, CHUNK)
    batch3 = batch.reshape(N // 1000, 1, 1000)

    a1 = _sc_segment_sum(x, src_r, dst_r)
    h1 = _tc_affine([(a1[0], W1_rel), (a1[1], W1_rel), (x, W1_root)], b1, True)

    a2 = _sc_segment_sum(h1, src_r, dst_r)
    h2a, h2b = _tc_affine([(a2[0], W2_rel), (a2[1], W2_rel), (h1, W2_root)],
                          b2, True, out_splits=[128, 128])

    dst_f = edge_index[1].reshape(NW, EPW)
    zeros = jnp.zeros((CHUNK, 128), jnp.float32)
    w = _sc_hist(src_r, dst_f, batch, zeros)
    return _tc_pool_fused(h2a, h2b, w[0], w[1], batch3, W3_rel, W3_root, b3)


# speedup vs baseline: 10.8765x; 1.1964x over previous
import dataclasses
import functools

import jax
import jax.numpy as jnp
from jax import lax
from jax.experimental import pallas as pl
from jax.experimental.pallas import tpu as pltpu
from jax.experimental.pallas import tpu_sc as plsc

N = 10000
E = 320000
G = 64

NC, NS = 2, 16
NW = NC * NS
EPW = E // NW
CHUNK = 80
NCHUNK = EPW // CHUNK
HCHUNK = 80
HNCHUNK = EPW // HCHUNK
NPAD = 10240
RPS = NPAD // NS
ZROWS = 16


def _sc_segment_sum(h, src_r, dst_r, zeros):
    mesh = plsc.VectorSubcoreMesh(core_axis_name="c", subcore_axis_name="s")

    @functools.partial(
        pl.kernel,
        out_type=jax.ShapeDtypeStruct((NC, NPAD, 128), jnp.float32),
        mesh=mesh,
        scratch_types=[
            pltpu.VMEM((NCHUNK, CHUNK), jnp.int32),
            pltpu.VMEM((2, CHUNK), jnp.int32),
            pltpu.VMEM((2, CHUNK, 128), jnp.float32),
            pltpu.VMEM_SHARED((NPAD, 128), jnp.float32),
            pltpu.SemaphoreType.DMA,
            pltpu.SemaphoreType.DMA,
            pltpu.SemaphoreType.DMA,
        ],
    )
    def k(h_hbm, src_hbm, dst_hbm, zeros_hbm, out_hbm,
          src_v, dst_w, rows_v, agg_sh, sg, ss, si):
        cid = lax.axis_index("c")
        sid = lax.axis_index("s")
        wid = cid * NS + sid

        pltpu.sync_copy(zeros_hbm.at[pl.ds(0, CHUNK)], rows_v.at[0])
        @pl.loop(0, RPS, step=CHUNK)
        def _(r0):
            pltpu.sync_copy(rows_v.at[0], agg_sh.at[pl.ds(sid * RPS + r0, CHUNK)])

        pltpu.sync_copy(src_hbm.at[wid], src_v)

        plsc.subcore_barrier()

        def gather(j, slot):
            return pltpu.async_copy(h_hbm.at[src_v.at[j]], rows_v.at[slot], sg)

        def gather_wait(j, slot):
            pltpu.make_async_copy(h_hbm.at[src_v.at[j]], rows_v.at[slot],
                                  sg).wait()

        def scat(j, slot):
            return pltpu.async_copy(rows_v.at[slot], agg_sh.at[dst_w.at[slot]],
                                    ss, add=True)

        def scat_wait(j, slot):
            pltpu.make_async_copy(rows_v.at[slot], agg_sh.at[dst_w.at[slot]],
                                  ss).wait()

        def icopy(j, slot):
            return pltpu.async_copy(dst_hbm.at[wid, j], dst_w.at[slot], si)

        def iwait(j, slot):
            pltpu.make_async_copy(dst_hbm.at[wid, j], dst_w.at[slot], si).wait()

        icopy(0, 0)
        gather(0, 0)

        @pl.loop(0, NCHUNK)
        def _(j):
            slot = lax.rem(j, 2)
            other = 1 - slot

            gather_wait(j, slot)

            @pl.when(j >= 1)
            def _():
                scat_wait(j - 1, other)

            iwait(j, slot)

            @pl.when(j + 1 < NCHUNK)
            def _():
                icopy(j + 1, other)
                gather(j + 1, other)

            scat(j, slot)

        scat_wait(NCHUNK - 1, lax.rem(NCHUNK - 1, 2))

        plsc.subcore_barrier()

        pltpu.sync_copy(agg_sh.at[pl.ds(sid * RPS, RPS)],
                        out_hbm.at[cid, pl.ds(sid * RPS, RPS)])

    return k(h, src_r, dst_r, zeros)


def _sc_hist(hsrc_r, dst_f, batch, zeros):
    mesh = plsc.VectorSubcoreMesh(core_axis_name="c", subcore_axis_name="s")
    cp = pltpu.CompilerParams()
    if "needs_layout_passes" in pltpu.CompilerParams.__dataclass_fields__:
        cp = dataclasses.replace(cp, needs_layout_passes=False)

    @functools.partial(
        pl.kernel,
        out_type=jax.ShapeDtypeStruct((NC, NPAD, 128), jnp.float32),
        mesh=mesh,
        compiler_params=cp,
        scratch_types=[
            pltpu.VMEM((HNCHUNK, HCHUNK), jnp.int32),
            pltpu.VMEM((EPW,), jnp.int32),
            pltpu.VMEM((N,), jnp.int32),
            pltpu.VMEM((HCHUNK, 128), jnp.float32),
            pltpu.VMEM_SHARED((NPAD, 128), jnp.float32),
        ],
    )
    def k(src_hbm, dst_hbm, batch_hbm, zeros_hbm, out_hbm,
          src_v, dst_v, batch_v, oh_v, w_sh):
        cid = lax.axis_index("c")
        sid = lax.axis_index("s")
        wid = cid * NS + sid

        pltpu.sync_copy(zeros_hbm.at[pl.ds(0, HCHUNK)], oh_v)

        @pl.loop(0, RPS, step=HCHUNK)
        def _(r0):
            pltpu.sync_copy(oh_v, w_sh.at[pl.ds(sid * RPS + r0, HCHUNK)])

        pltpu.sync_copy(src_hbm.at[wid], src_v)
        pltpu.sync_copy(dst_hbm.at[wid], dst_v)
        pltpu.sync_copy(batch_hbm, batch_v)

        plsc.subcore_barrier()

        ones16 = jnp.ones((16,), jnp.float32)
        zeros16 = jnp.zeros((16,), jnp.float32)
        iota16 = lax.iota(jnp.int32, 16)

        @pl.loop(0, HNCHUNK)
        def _(j):
            cols = []
            for k16 in range(HCHUNK // 16):
                off = j * HCHUNK + 16 * k16
                d16 = dst_v.at[pl.ds(off, 16)][...]
                g16 = plsc.load_gather(batch_v, [d16])
                rows = iota16 + 16 * k16
                plsc.addupdate_scatter(oh_v, [rows, g16], ones16)
                cols.append((rows, g16))
            pltpu.sync_copy(oh_v, w_sh.at[src_v.at[j]], add=True)
            for rows, g16 in cols:
                plsc.store_scatter(oh_v, [rows, g16], zeros16)

        plsc.subcore_barrier()

        pltpu.sync_copy(w_sh.at[pl.ds(sid * RPS, RPS)],
                        out_hbm.at[cid, pl.ds(sid * RPS, RPS)])

    return k(hsrc_r, dst_f, batch, zeros)


def _tc_affine(parts, b, relu, out_splits=None):
    dout = b.shape[0]
    blk = 1000
    b2 = b.reshape(1, dout)
    nparts = len(parts)
    splits = out_splits or [dout]
    assert sum(splits) == dout

    def body(*refs):
        o_refs = refs[nparts * 2 + 1:]
        b_ref = refs[nparts * 2]
        acc = jnp.broadcast_to(b_ref[...], (blk, dout))
        for i in range(nparts):
            a = refs[2 * i][...]
            w = refs[2 * i + 1][...]
            acc = acc + lax.dot_general(a, w, (((1,), (0,)), ((), ())),
                                        precision=lax.Precision.HIGHEST,
                                        preferred_element_type=jnp.float32)
        if relu:
            acc = jnp.maximum(acc, 0.0)
        c0 = 0
        for o_ref, w in zip(o_refs, splits):
            o_ref[...] = acc[:, c0:c0 + w]
            c0 += w

    in_specs = []
    args = []
    for a, w in parts:
        kk = a.shape[1]
        in_specs.append(pl.BlockSpec((blk, kk), lambda i: (i, 0)))
        in_specs.append(pl.BlockSpec((kk, dout), lambda i: (0, 0)))
        args.extend([a, w])
    in_specs.append(pl.BlockSpec((1, dout), lambda i: (0, 0)))
    args.append(b2)

    out = pl.pallas_call(
        body,
        grid=(N // blk,),
        in_specs=in_specs,
        out_specs=[pl.BlockSpec((blk, w), lambda i: (i, 0)) for w in splits],
        out_shape=[jax.ShapeDtypeStruct((N, w), jnp.float32) for w in splits],
    )(*args)
    return out[0] if out_splits is None else out


def _tc_pool_fused(h2a, h2b, w0, w1, batch3, W3_rel, W3_root, b3):
    blk = 1000
    nb = N // blk

    def body(ha_ref, hb_ref, w0_ref, w1_ref, b_ref, wr_ref, wt_ref, b3_ref,
             o_ref, S, B, cnts):
        i = pl.program_id(0)
        bb = b_ref[0, 0, :]
        oh = (bb[:, None] == lax.broadcasted_iota(jnp.int32, (blk, G), 1))
        oh = oh.astype(jnp.float32)
        ws = w0_ref[...][:, :G] + w1_ref[...][:, :G]

        def mmT(lhs, rhs):
            return lax.dot_general(lhs, rhs, (((0,), (0,)), ((), ())),
                                   precision=lax.Precision.HIGHEST,
                                   preferred_element_type=jnp.float32)

        sa = mmT(oh, ha_ref[...])
        sb = mmT(oh, hb_ref[...])
        ba = mmT(ws, ha_ref[...])
        bb2 = mmT(ws, hb_ref[...])
        pcnt = jnp.sum(oh, axis=0).reshape(1, G)

        @pl.when(i == 0)
        def _():
            S[:, :128] = sa
            S[:, 128:] = sb
            B[:, :128] = ba
            B[:, 128:] = bb2
            cnts[...] = pcnt

        @pl.when(i > 0)
        def _():
            S[:, :128] += sa
            S[:, 128:] += sb
            B[:, :128] += ba
            B[:, 128:] += bb2
            cnts[...] += pcnt

        @pl.when(i == nb - 1)
        def _():
            cnt = cnts[...].reshape(G, 1)
            psum = lax.dot_general(B[...], wr_ref[...], (((1,), (0,)), ((), ())),
                                   precision=lax.Precision.HIGHEST,
                                   preferred_element_type=jnp.float32)
            psum += lax.dot_general(S[...], wt_ref[...], (((1,), (0,)), ((), ())),
                                    precision=lax.Precision.HIGHEST,
                                    preferred_element_type=jnp.float32)
            psum += cnt * b3_ref[...]
            pooled = psum / jnp.maximum(cnt, 1.0)
            nrm = jnp.sqrt(jnp.sum(pooled * pooled, axis=1, keepdims=True))
            o_ref[...] = pooled / jnp.maximum(nrm, 1e-12)

    return pl.pallas_call(
        body,
        grid=(nb,),
        in_specs=[pl.BlockSpec((blk, 128), lambda i: (i, 0)),
                  pl.BlockSpec((blk, 128), lambda i: (i, 0)),
                  pl.BlockSpec((blk, 128), lambda i: (i, 0)),
                  pl.BlockSpec((blk, 128), lambda i: (i, 0)),
                  pl.BlockSpec((1, 1, blk), lambda i: (i, 0, 0)),
                  pl.BlockSpec((256, 512), lambda i: (0, 0)),
                  pl.BlockSpec((256, 512), lambda i: (0, 0)),
                  pl.BlockSpec((1, 512), lambda i: (0, 0))],
        out_specs=pl.BlockSpec((G, 512), lambda i: (0, 0)),
        out_shape=jax.ShapeDtypeStruct((G, 512), jnp.float32),
        scratch_shapes=[pltpu.VMEM((G, 256), jnp.float32),
                        pltpu.VMEM((G, 256), jnp.float32),
                        pltpu.VMEM((1, G), jnp.float32)],
    )(h2a, h2b, w0, w1, batch3, W3_rel, W3_root, b3.reshape(1, 512))


def kernel(x, edge_index, batch, W1_rel, b1, W1_root, W2_rel, b2, W2_root,
           W3_rel, b3, W3_root):
    src_r = edge_index[0].reshape(NW, NCHUNK, CHUNK)
    dst_r = edge_index[1].reshape(NW, NCHUNK, CHUNK)
    batch3 = batch.reshape(N // 1000, 1, 1000)
    zeros = jnp.zeros((HCHUNK, 128), jnp.float32)

    a1 = _sc_segment_sum(x, src_r, dst_r, zeros)
    h1 = _tc_affine([(a1[0], W1_rel), (a1[1], W1_rel), (x, W1_root)], b1, True)

    a2 = _sc_segment_sum(h1, src_r, dst_r, zeros)
    h2a, h2b = _tc_affine([(a2[0], W2_rel), (a2[1], W2_rel), (h1, W2_root)],
                          b2, True, out_splits=[128, 128])

    dst_f = edge_index[1].reshape(NW, EPW)
    hsrc_r = edge_index[0].reshape(NW, HNCHUNK, HCHUNK)
    w = _sc_hist(hsrc_r, dst_f, batch, zeros)
    return _tc_pool_fused(h2a, h2b, w[0], w[1], batch3, W3_rel, W3_root, b3)


# speedup vs baseline: 13.7991x; 1.2687x over previous
import dataclasses
import functools

import jax
import jax.numpy as jnp
from jax import lax
from jax.experimental import pallas as pl
from jax.experimental.pallas import tpu as pltpu
from jax.experimental.pallas import tpu_sc as plsc

N = 10000
E = 320000
G = 64

NC, NS = 2, 16
NW = NC * NS
EPW = E // NW
CHUNK = 80
NCHUNK = EPW // CHUNK
HCHUNK = 80
HNCHUNK = EPW // HCHUNK
NPAD = 10240
RPS = NPAD // NS
ZROWS = 16


def _sc_segment_sum(h, eix, zeros):
    mesh = plsc.VectorSubcoreMesh(core_axis_name="c", subcore_axis_name="s")

    @functools.partial(
        pl.kernel,
        out_type=jax.ShapeDtypeStruct((NC, NPAD, 128), jnp.float32),
        mesh=mesh,
        scratch_types=[
            pltpu.VMEM((4, 2, CHUNK), jnp.int32),
            pltpu.VMEM((3, CHUNK, 128), jnp.float32),
            pltpu.VMEM_SHARED((NPAD, 128), jnp.float32),
            pltpu.SemaphoreType.DMA((3,)),
            pltpu.SemaphoreType.DMA,
            pltpu.SemaphoreType.DMA((4,)),
        ],
    )
    def k(h_hbm, eix_hbm, zeros_hbm, out_hbm,
          win_v, rows_v, agg_sh, sg, ss, si):
        cid = lax.axis_index("c")
        sid = lax.axis_index("s")
        wid = cid * NS + sid

        pltpu.sync_copy(zeros_hbm.at[pl.ds(0, CHUNK)], rows_v.at[0])
        @pl.loop(0, RPS, step=CHUNK)
        def _(r0):
            pltpu.sync_copy(rows_v.at[0], agg_sh.at[pl.ds(sid * RPS + r0, CHUNK)])

        plsc.subcore_barrier()

        def icopy(j, w):
            return pltpu.async_copy(eix_hbm.at[wid, j], win_v.at[w], si.at[w])

        def iwait(j, w):
            pltpu.make_async_copy(eix_hbm.at[wid, j], win_v.at[w],
                                  si.at[w]).wait()

        def gather(j, r, w):
            return pltpu.async_copy(h_hbm.at[win_v.at[w, 0]], rows_v.at[r],
                                    sg.at[r])

        def gather_wait(j, r, w):
            pltpu.make_async_copy(h_hbm.at[win_v.at[w, 0]], rows_v.at[r],
                                  sg.at[r]).wait()

        def scat(j, r, w):
            return pltpu.async_copy(rows_v.at[r], agg_sh.at[win_v.at[w, 1]],
                                    ss, add=True)

        def scat_wait(j, r, w):
            pltpu.make_async_copy(rows_v.at[r], agg_sh.at[win_v.at[w, 1]],
                                  ss).wait()

        icopy(0, 0)
        icopy(1, 1)
        icopy(2, 2)
        iwait(0, 0)
        gather(0, 0, 0)
        iwait(1, 1)
        gather(1, 1, 1)

        @pl.loop(0, NCHUNK)
        def _(j):
            r = lax.rem(j, 3)
            w = lax.rem(j, 4)
            r2 = lax.rem(j + 2, 3)
            w2 = lax.rem(j + 2, 4)
            w3 = lax.rem(j + 3, 4)

            @pl.when(j >= 1)
            def _():
                scat_wait(j - 1, lax.rem(j - 1, 3), lax.rem(j - 1, 4))

            @pl.when(j + 3 < NCHUNK)
            def _():
                icopy(j + 3, w3)

            @pl.when(j + 2 < NCHUNK)
            def _():
                iwait(j + 2, w2)
                gather(j + 2, r2, w2)

            gather_wait(j, r, w)
            scat(j, r, w)

        scat_wait(NCHUNK - 1, lax.rem(NCHUNK - 1, 3), lax.rem(NCHUNK - 1, 4))

        plsc.subcore_barrier()

        pltpu.sync_copy(agg_sh.at[pl.ds(sid * RPS, RPS)],
                        out_hbm.at[cid, pl.ds(sid * RPS, RPS)])

    return k(h, eix, zeros)


def _sc_hist(hsrc_r, dst_f, batch, zeros):
    mesh = plsc.VectorSubcoreMesh(core_axis_name="c", subcore_axis_name="s")
    cp = pltpu.CompilerParams()
    if "needs_layout_passes" in pltpu.CompilerParams.__dataclass_fields__:
        cp = dataclasses.replace(cp, needs_layout_passes=False)

    @functools.partial(
        pl.kernel,
        out_type=jax.ShapeDtypeStruct((NC, NPAD, 128), jnp.float32),
        mesh=mesh,
        compiler_params=cp,
        scratch_types=[
            pltpu.VMEM((HNCHUNK, HCHUNK), jnp.int32),
            pltpu.VMEM((EPW,), jnp.int32),
            pltpu.VMEM((N,), jnp.int32),
            pltpu.VMEM((HCHUNK, 128), jnp.float32),
            pltpu.VMEM_SHARED((NPAD, 128), jnp.float32),
        ],
    )
    def k(src_hbm, dst_hbm, batch_hbm, zeros_hbm, out_hbm,
          src_v, dst_v, batch_v, oh_v, w_sh):
        cid = lax.axis_index("c")
        sid = lax.axis_index("s")
        wid = cid * NS + sid

        pltpu.sync_copy(zeros_hbm.at[pl.ds(0, HCHUNK)], oh_v)

        @pl.loop(0, RPS, step=HCHUNK)
        def _(r0):
            pltpu.sync_copy(oh_v, w_sh.at[pl.ds(sid * RPS + r0, HCHUNK)])

        pltpu.sync_copy(src_hbm.at[wid], src_v)
        pltpu.sync_copy(dst_hbm.at[wid], dst_v)
        pltpu.sync_copy(batch_hbm, batch_v)

        plsc.subcore_barrier()

        ones16 = jnp.ones((16,), jnp.float32)
        zeros16 = jnp.zeros((16,), jnp.float32)
        iota16 = lax.iota(jnp.int32, 16)

        @pl.loop(0, HNCHUNK)
        def _(j):
            cols = []
            for k16 in range(HCHUNK // 16):
                off = j * HCHUNK + 16 * k16
                d16 = dst_v.at[pl.ds(off, 16)][...]
                g16 = plsc.load_gather(batch_v, [d16])
                rows = iota16 + 16 * k16
                plsc.addupdate_scatter(oh_v, [rows, g16], ones16)
                cols.append((rows, g16))
            pltpu.sync_copy(oh_v, w_sh.at[src_v.at[j]], add=True)
            for rows, g16 in cols:
                plsc.store_scatter(oh_v, [rows, g16], zeros16)

        plsc.subcore_barrier()

        pltpu.sync_copy(w_sh.at[pl.ds(sid * RPS, RPS)],
                        out_hbm.at[cid, pl.ds(sid * RPS, RPS)])

    return k(hsrc_r, dst_f, batch, zeros)


def _tc_affine(parts, b, relu, out_splits=None):
    dout = b.shape[0]
    blk = 1000
    b2 = b.reshape(1, dout)
    nparts = len(parts)
    splits = out_splits or [dout]
    assert sum(splits) == dout

    def body(*refs):
        o_refs = refs[nparts * 2 + 1:]
        b_ref = refs[nparts * 2]
        acc = jnp.broadcast_to(b_ref[...], (blk, dout))
        for i in range(nparts):
            a = refs[2 * i][...]
            w = refs[2 * i + 1][...]
            acc = acc + lax.dot_general(a, w, (((1,), (0,)), ((), ())),
                                        precision=lax.Precision.HIGHEST,
                                        preferred_element_type=jnp.float32)
        if relu:
            acc = jnp.maximum(acc, 0.0)
        c0 = 0
        for o_ref, w in zip(o_refs, splits):
            o_ref[...] = acc[:, c0:c0 + w]
            c0 += w

    in_specs = []
    args = []
    for a, w in parts:
        kk = a.shape[1]
        in_specs.append(pl.BlockSpec((blk, kk), lambda i: (i, 0)))
        in_specs.append(pl.BlockSpec((kk, dout), lambda i: (0, 0)))
        args.extend([a, w])
    in_specs.append(pl.BlockSpec((1, dout), lambda i: (0, 0)))
    args.append(b2)

    out = pl.pallas_call(
        body,
        grid=(N // blk,),
        in_specs=in_specs,
        out_specs=[pl.BlockSpec((blk, w), lambda i: (i, 0)) for w in splits],
        out_shape=[jax.ShapeDtypeStruct((N, w), jnp.float32) for w in splits],
    )(*args)
    return out[0] if out_splits is None else out


def _tc_pool_fused(h2a, h2b, w0, w1, batch3, W3_rel, W3_root, b3):
    blk = 1000
    nb = N // blk

    def body(ha_ref, hb_ref, w0_ref, w1_ref, b_ref, wr_ref, wt_ref, b3_ref,
             o_ref, S, B, cnts):
        i = pl.program_id(0)
        bb = b_ref[0, 0, :]
        oh = (bb[:, None] == lax.broadcasted_iota(jnp.int32, (blk, G), 1))
        oh = oh.astype(jnp.float32)
        ws = w0_ref[...][:, :G] + w1_ref[...][:, :G]

        def mmT(lhs, rhs):
            return lax.dot_general(lhs, rhs, (((0,), (0,)), ((), ())),
                                   precision=lax.Precision.HIGHEST,
                                   preferred_element_type=jnp.float32)

        sa = mmT(oh, ha_ref[...])
        sb = mmT(oh, hb_ref[...])
        ba = mmT(ws, ha_ref[...])
        bb2 = mmT(ws, hb_ref[...])
        pcnt = jnp.sum(oh, axis=0).reshape(1, G)

        @pl.when(i == 0)
        def _():
            S[:, :128] = sa
            S[:, 128:] = sb
            B[:, :128] = ba
            B[:, 128:] = bb2
            cnts[...] = pcnt

        @pl.when(i > 0)
        def _():
            S[:, :128] += sa
            S[:, 128:] += sb
            B[:, :128] += ba
            B[:, 128:] += bb2
            cnts[...] += pcnt

        @pl.when(i == nb - 1)
        def _():
            cnt = cnts[...].reshape(G, 1)
            psum = lax.dot_general(B[...], wr_ref[...], (((1,), (0,)), ((), ())),
                                   precision=lax.Precision.HIGHEST,
                                   preferred_element_type=jnp.float32)
            psum += lax.dot_general(S[...], wt_ref[...], (((1,), (0,)), ((), ())),
                                    precision=lax.Precision.HIGHEST,
                                    preferred_element_type=jnp.float32)
            psum += cnt * b3_ref[...]
            pooled = psum / jnp.maximum(cnt, 1.0)
            nrm = jnp.sqrt(jnp.sum(pooled * pooled, axis=1, keepdims=True))
            o_ref[...] = pooled / jnp.maximum(nrm, 1e-12)

    return pl.pallas_call(
        body,
        grid=(nb,),
        in_specs=[pl.BlockSpec((blk, 128), lambda i: (i, 0)),
                  pl.BlockSpec((blk, 128), lambda i: (i, 0)),
                  pl.BlockSpec((blk, 128), lambda i: (i, 0)),
                  pl.BlockSpec((blk, 128), lambda i: (i, 0)),
                  pl.BlockSpec((1, 1, blk), lambda i: (i, 0, 0)),
                  pl.BlockSpec((256, 512), lambda i: (0, 0)),
                  pl.BlockSpec((256, 512), lambda i: (0, 0)),
                  pl.BlockSpec((1, 512), lambda i: (0, 0))],
        out_specs=pl.BlockSpec((G, 512), lambda i: (0, 0)),
        out_shape=jax.ShapeDtypeStruct((G, 512), jnp.float32),
        scratch_shapes=[pltpu.VMEM((G, 256), jnp.float32),
                        pltpu.VMEM((G, 256), jnp.float32),
                        pltpu.VMEM((1, G), jnp.float32)],
    )(h2a, h2b, w0, w1, batch3, W3_rel, W3_root, b3.reshape(1, 512))


def kernel(x, edge_index, batch, W1_rel, b1, W1_root, W2_rel, b2, W2_root,
           W3_rel, b3, W3_root):
    src_r = edge_index[0].reshape(NW, NCHUNK, CHUNK)
    dst_r = edge_index[1].reshape(NW, NCHUNK, CHUNK)
    eix = jnp.stack([src_r, dst_r], axis=2)
    batch3 = batch.reshape(N // 1000, 1, 1000)
    zeros = jnp.zeros((HCHUNK, 128), jnp.float32)

    a1 = _sc_segment_sum(x, eix, zeros)
    h1 = _tc_affine([(a1[0], W1_rel), (a1[1], W1_rel), (x, W1_root)], b1, True)

    a2 = _sc_segment_sum(h1, eix, zeros)
    h2a, h2b = _tc_affine([(a2[0], W2_rel), (a2[1], W2_rel), (h1, W2_root)],
                          b2, True, out_splits=[128, 128])

    dst_f = edge_index[1].reshape(NW, EPW)
    hsrc_r = edge_index[0].reshape(NW, HNCHUNK, HCHUNK)
    w = _sc_hist(hsrc_r, dst_f, batch, zeros)
    return _tc_pool_fused(h2a, h2b, w[0], w[1], batch3, W3_rel, W3_root, b3)


# speedup vs baseline: 14.2272x; 1.0310x over previous
import dataclasses
import functools

import jax
import jax.numpy as jnp
from jax import lax
from jax.experimental import pallas as pl
from jax.experimental.pallas import tpu as pltpu
from jax.experimental.pallas import tpu_sc as plsc

N = 10000
E = 320000
G = 64

NC, NS = 2, 16
NW = NC * NS
EPW = E // NW
CHUNK = 80
NCHUNK = EPW // CHUNK
NPAD = 10240
RPS = NPAD // NS
ZROWS = 16


def _sc_segment_sum(h, eix, zeros):
    mesh = plsc.VectorSubcoreMesh(core_axis_name="c", subcore_axis_name="s")

    @functools.partial(
        pl.kernel,
        out_type=jax.ShapeDtypeStruct((NC, NPAD, 128), jnp.float32),
        mesh=mesh,
        scratch_types=[
            pltpu.VMEM((4, 2, CHUNK), jnp.int32),
            pltpu.VMEM((3, CHUNK, 128), jnp.float32),
            pltpu.VMEM_SHARED((NPAD, 128), jnp.float32),
            pltpu.SemaphoreType.DMA((3,)),
            pltpu.SemaphoreType.DMA,
            pltpu.SemaphoreType.DMA((4,)),
        ],
    )
    def k(h_hbm, eix_hbm, zeros_hbm, out_hbm,
          win_v, rows_v, agg_sh, sg, ss, si):
        cid = lax.axis_index("c")
        sid = lax.axis_index("s")
        wid = cid * NS + sid

        pltpu.sync_copy(zeros_hbm.at[pl.ds(0, CHUNK)], rows_v.at[0])
        @pl.loop(0, RPS, step=CHUNK)
        def _(r0):
            pltpu.sync_copy(rows_v.at[0], agg_sh.at[pl.ds(sid * RPS + r0, CHUNK)])

        plsc.subcore_barrier()

        def icopy(j, w):
            return pltpu.async_copy(eix_hbm.at[wid, j], win_v.at[w], si.at[w])

        def iwait(j, w):
            pltpu.make_async_copy(eix_hbm.at[wid, j], win_v.at[w],
                                  si.at[w]).wait()

        def gather(j, r, w):
            return pltpu.async_copy(h_hbm.at[win_v.at[w, 0]], rows_v.at[r],
                                    sg.at[r])

        def gather_wait(j, r, w):
            pltpu.make_async_copy(h_hbm.at[win_v.at[w, 0]], rows_v.at[r],
                                  sg.at[r]).wait()

        def scat(j, r, w):
            return pltpu.async_copy(rows_v.at[r], agg_sh.at[win_v.at[w, 1]],
                                    ss, add=True)

        def scat_wait(j, r, w):
            pltpu.make_async_copy(rows_v.at[r], agg_sh.at[win_v.at[w, 1]],
                                  ss).wait()

        icopy(0, 0)
        icopy(1, 1)
        icopy(2, 2)
        iwait(0, 0)
        gather(0, 0, 0)
        iwait(1, 1)
        gather(1, 1, 1)

        @pl.loop(0, NCHUNK)
        def _(j):
            r = lax.rem(j, 3)
            w = lax.rem(j, 4)
            r2 = lax.rem(j + 2, 3)
            w2 = lax.rem(j + 2, 4)
            w3 = lax.rem(j + 3, 4)

            @pl.when(j >= 1)
            def _():
                scat_wait(j - 1, lax.rem(j - 1, 3), lax.rem(j - 1, 4))

            @pl.when(j + 3 < NCHUNK)
            def _():
                icopy(j + 3, w3)

            @pl.when(j + 2 < NCHUNK)
            def _():
                iwait(j + 2, w2)
                gather(j + 2, r2, w2)

            gather_wait(j, r, w)
            scat(j, r, w)

        scat_wait(NCHUNK - 1, lax.rem(NCHUNK - 1, 3), lax.rem(NCHUNK - 1, 4))

        plsc.subcore_barrier()

        pltpu.sync_copy(agg_sh.at[pl.ds(sid * RPS, RPS)],
                        out_hbm.at[cid, pl.ds(sid * RPS, RPS)])

    return k(h, eix, zeros)


def _sc_hist(eix, batch, zeros):
    mesh = plsc.VectorSubcoreMesh(core_axis_name="c", subcore_axis_name="s")
    cp = pltpu.CompilerParams()
    if "needs_layout_passes" in pltpu.CompilerParams.__dataclass_fields__:
        cp = dataclasses.replace(cp, needs_layout_passes=False)

    @functools.partial(
        pl.kernel,
        out_type=jax.ShapeDtypeStruct((NC, NPAD, 128), jnp.float32),
        mesh=mesh,
        compiler_params=cp,
        scratch_types=[
            pltpu.VMEM((4, 2, CHUNK), jnp.int32),
            pltpu.VMEM((N,), jnp.int32),
            pltpu.VMEM((2, CHUNK, 128), jnp.float32),
            pltpu.VMEM((2, CHUNK), jnp.int32),
            pltpu.VMEM_SHARED((NPAD, 128), jnp.float32),
            pltpu.SemaphoreType.DMA((2,)),
            pltpu.SemaphoreType.DMA((4,)),
        ],
    )
    def k(eix_hbm, batch_hbm, zeros_hbm, out_hbm,
          win_v, batch_v, oh_v, colb_v, w_sh, ss, si):
        cid = lax.axis_index("c")
        sid = lax.axis_index("s")
        wid = cid * NS + sid

        pltpu.sync_copy(zeros_hbm, oh_v.at[0])
        pltpu.sync_copy(zeros_hbm, oh_v.at[1])

        @pl.loop(0, RPS, step=CHUNK)
        def _(r0):
            pltpu.sync_copy(oh_v.at[0], w_sh.at[pl.ds(sid * RPS + r0, CHUNK)])

        pltpu.sync_copy(batch_hbm, batch_v)

        def icopy(j, w):
            return pltpu.async_copy(eix_hbm.at[wid, j], win_v.at[w], si.at[w])

        def iwait(j, w):
            pltpu.make_async_copy(eix_hbm.at[wid, j], win_v.at[w],
                                  si.at[w]).wait()

        def stream(j, b, w):
            return pltpu.async_copy(oh_v.at[b], w_sh.at[win_v.at[w, 0]],
                                    ss.at[b], add=True)

        def swait(j, b, w):
            pltpu.make_async_copy(oh_v.at[b], w_sh.at[win_v.at[w, 0]],
                                  ss.at[b]).wait()

        plsc.subcore_barrier()

        icopy(0, 0)
        icopy(1, 1)
        icopy(2, 2)
        icopy(3, 3)

        ones16 = jnp.ones((16,), jnp.float32)
        zeros16 = jnp.zeros((16,), jnp.float32)
        iota16 = lax.iota(jnp.int32, 16)

        @pl.loop(0, NCHUNK)
        def _(j):
            b = lax.rem(j, 2)
            w = lax.rem(j, 4)

            @pl.when(j >= 2)
            def _():
                swait(j - 2, b, lax.rem(j - 2, 4))
                for k16 in range(CHUNK // 16):
                    g16 = colb_v.at[b, pl.ds(16 * k16, 16)][...]
                    plsc.store_scatter(oh_v.at[b],
                                       [iota16 + 16 * k16, g16], zeros16)

                @pl.when(j + 2 < NCHUNK)
                def _():
                    icopy(j + 2, lax.rem(j + 2, 4))

            iwait(j, w)
            for k16 in range(CHUNK // 16):
                d16 = win_v.at[w, 1, pl.ds(16 * k16, 16)][...]
                g16 = plsc.load_gather(batch_v, [d16])
                plsc.addupdate_scatter(oh_v.at[b],
                                       [iota16 + 16 * k16, g16], ones16)
                colb_v.at[b, pl.ds(16 * k16, 16)][...] = g16
            stream(j, b, w)

        swait(NCHUNK - 2, lax.rem(NCHUNK - 2, 2), lax.rem(NCHUNK - 2, 4))
        swait(NCHUNK - 1, lax.rem(NCHUNK - 1, 2), lax.rem(NCHUNK - 1, 4))

        plsc.subcore_barrier()

        pltpu.sync_copy(w_sh.at[pl.ds(sid * RPS, RPS)],
                        out_hbm.at[cid, pl.ds(sid * RPS, RPS)])

    return k(eix, batch, zeros)


def _tc_affine(parts, b, relu, out_splits=None):
    dout = b.shape[0]
    blk = 1000
    b2 = b.reshape(1, dout)
    nparts = len(parts)
    splits = out_splits or [dout]
    assert sum(splits) == dout

    def body(*refs):
        o_refs = refs[nparts * 2 + 1:]
        b_ref = refs[nparts * 2]
        acc = jnp.broadcast_to(b_ref[...], (blk, dout))
        for i in range(nparts):
            a = refs[2 * i][...]
            w = refs[2 * i + 1][...]
            acc = acc + lax.dot_general(a, w, (((1,), (0,)), ((), ())),
                                        precision=lax.Precision.HIGHEST,
                                        preferred_element_type=jnp.float32)
        if relu:
            acc = jnp.maximum(acc, 0.0)
        c0 = 0
        for o_ref, w in zip(o_refs, splits):
            o_ref[...] = acc[:, c0:c0 + w]
            c0 += w

    in_specs = []
    args = []
    for a, w in parts:
        kk = a.shape[1]
        in_specs.append(pl.BlockSpec((blk, kk), lambda i: (i, 0)))
        in_specs.append(pl.BlockSpec((kk, dout), lambda i: (0, 0)))
        args.extend([a, w])
    in_specs.append(pl.BlockSpec((1, dout), lambda i: (0, 0)))
    args.append(b2)

    out = pl.pallas_call(
        body,
        grid=(N // blk,),
        in_specs=in_specs,
        out_specs=[pl.BlockSpec((blk, w), lambda i: (i, 0)) for w in splits],
        out_shape=[jax.ShapeDtypeStruct((N, w), jnp.float32) for w in splits],
    )(*args)
    return out[0] if out_splits is None else out


def _tc_pool_fused(h2a, h2b, w0, w1, batch3, W3_rel, W3_root, b3):
    blk = 1000
    nb = N // blk

    def body(ha_ref, hb_ref, w0_ref, w1_ref, b_ref, wr_ref, wt_ref, b3_ref,
             o_ref, S, B, cnts):
        i = pl.program_id(0)
        bb = b_ref[0, 0, :]
        oh = (bb[:, None] == lax.broadcasted_iota(jnp.int32, (blk, G), 1))
        oh = oh.astype(jnp.float32)
        ws = w0_ref[...][:, :G] + w1_ref[...][:, :G]

        def mmT(lhs, rhs):
            return lax.dot_general(lhs, rhs, (((0,), (0,)), ((), ())),
                                   precision=lax.Precision.HIGHEST,
                                   preferred_element_type=jnp.float32)

        sa = mmT(oh, ha_ref[...])
        sb = mmT(oh, hb_ref[...])
        ba = mmT(ws, ha_ref[...])
        bb2 = mmT(ws, hb_ref[...])
        pcnt = jnp.sum(oh, axis=0).reshape(1, G)

        @pl.when(i == 0)
        def _():
            S[:, :128] = sa
            S[:, 128:] = sb
            B[:, :128] = ba
            B[:, 128:] = bb2
            cnts[...] = pcnt

        @pl.when(i > 0)
        def _():
            S[:, :128] += sa
            S[:, 128:] += sb
            B[:, :128] += ba
            B[:, 128:] += bb2
            cnts[...] += pcnt

        @pl.when(i == nb - 1)
        def _():
            cnt = cnts[...].reshape(G, 1)
            psum = lax.dot_general(B[...], wr_ref[...], (((1,), (0,)), ((), ())),
                                   precision=lax.Precision.HIGHEST,
                                   preferred_element_type=jnp.float32)
            psum += lax.dot_general(S[...], wt_ref[...], (((1,), (0,)), ((), ())),
                                    precision=lax.Precision.HIGHEST,
                                    preferred_element_type=jnp.float32)
            psum += cnt * b3_ref[...]
            pooled = psum / jnp.maximum(cnt, 1.0)
            nrm = jnp.sqrt(jnp.sum(pooled * pooled, axis=1, keepdims=True))
            o_ref[...] = pooled / jnp.maximum(nrm, 1e-12)

    return pl.pallas_call(
        body,
        grid=(nb,),
        in_specs=[pl.BlockSpec((blk, 128), lambda i: (i, 0)),
                  pl.BlockSpec((blk, 128), lambda i: (i, 0)),
                  pl.BlockSpec((blk, 128), lambda i: (i, 0)),
                  pl.BlockSpec((blk, 128), lambda i: (i, 0)),
                  pl.BlockSpec((1, 1, blk), lambda i: (i, 0, 0)),
                  pl.BlockSpec((256, 512), lambda i: (0, 0)),
                  pl.BlockSpec((256, 512), lambda i: (0, 0)),
                  pl.BlockSpec((1, 512), lambda i: (0, 0))],
        out_specs=pl.BlockSpec((G, 512), lambda i: (0, 0)),
        out_shape=jax.ShapeDtypeStruct((G, 512), jnp.float32),
        scratch_shapes=[pltpu.VMEM((G, 256), jnp.float32),
                        pltpu.VMEM((G, 256), jnp.float32),
                        pltpu.VMEM((1, G), jnp.float32)],
    )(h2a, h2b, w0, w1, batch3, W3_rel, W3_root, b3.reshape(1, 512))


def kernel(x, edge_index, batch, W1_rel, b1, W1_root, W2_rel, b2, W2_root,
           W3_rel, b3, W3_root):
    src_r = edge_index[0].reshape(NW, NCHUNK, CHUNK)
    dst_r = edge_index[1].reshape(NW, NCHUNK, CHUNK)
    eix = jnp.stack([src_r, dst_r], axis=2)
    batch3 = batch.reshape(N // 1000, 1, 1000)
    zeros = jnp.zeros((CHUNK, 128), jnp.float32)

    a1 = _sc_segment_sum(x, eix, zeros)
    h1 = _tc_affine([(a1[0], W1_rel), (a1[1], W1_rel), (x, W1_root)], b1, True)

    a2 = _sc_segment_sum(h1, eix, zeros)
    h2a, h2b = _tc_affine([(a2[0], W2_rel), (a2[1], W2_rel), (h1, W2_root)],
                          b2, True, out_splits=[128, 128])

    w = _sc_hist(eix, batch, zeros)
    return _tc_pool_fused(h2a, h2b, w[0], w[1], batch3, W3_rel, W3_root, b3)
